# baseline (speedup 1.0000x reference)
import functools

import jax
import jax.numpy as jnp
import numpy as np
from jax.experimental import pallas as pl
from jax.experimental.pallas import tpu as pltpu

_PMIN = np.float32(0.1)
_PMAX = np.float32(1.0)
_EPS_D = np.float32(1e-12)
_TINY = np.float32(np.finfo(np.float32).tiny)
_LANES = 128


def _stats_kernel(x_ref, out_ref, mn_ref, mx_ref, s1_ref, s2_ref, *,
                  n_cols, block_w):
    c = pl.program_id(0)
    nc = pl.num_programs(0)
    B = x_ref.shape[0]
    nfold = block_w // _LANES

    accmn = jnp.where(c == 0, jnp.full((B, _LANES), jnp.inf, jnp.float32),
                      mn_ref[...])
    accmx = jnp.where(c == 0, jnp.zeros((B, _LANES), jnp.float32),
                      mx_ref[...])
    accs1 = jnp.where(c == 0, jnp.zeros((B, _LANES), jnp.float32),
                      s1_ref[...])
    accs2 = jnp.where(c == 0, jnp.zeros((B, _LANES), jnp.float32),
                      s2_ref[...])

    base = c * block_w

    def fold(masked):
        amn, amx, a1, a2 = accmn, accmx, accs1, accs2
        lane = jax.lax.broadcasted_iota(jnp.int32, (B, _LANES), 1)
        for k in range(nfold):
            a = jnp.abs(x_ref[:, k * _LANES:(k + 1) * _LANES])
            if masked:
                valid = (base + k * _LANES) + lane < n_cols
                am = jnp.where(valid, a, 0.0)
                amn = jnp.minimum(amn, jnp.where(valid, a, jnp.inf))
            else:
                am = a
                amn = jnp.minimum(amn, a)
            amx = jnp.maximum(amx, am)
            a1 = a1 + am
            a2 = a2 + am * am
        mn_ref[...] = amn
        mx_ref[...] = amx
        s1_ref[...] = a1
        s2_ref[...] = a2

    if block_w * (n_cols // block_w) == n_cols:
        fold(False)
    else:
        @pl.when(c != nc - 1)
        def _():
            fold(False)

        @pl.when(c == nc - 1)
        def _():
            fold(True)

    @pl.when(c == nc - 1)
    def _():
        gmin = jnp.min(mn_ref[...], axis=-1, keepdims=True)
        gmax = jnp.max(mx_ref[...], axis=-1, keepdims=True)
        s1 = jnp.sum(s1_ref[...], axis=-1, keepdims=True)
        s2 = jnp.sum(s2_ref[...], axis=-1, keepdims=True)
        denom = gmax - gmin + _EPS_D
        inv_denom = 1.0 / denom
        nf = jnp.float32(n_cols)
        sumq = (s2 - 2.0 * gmin * s1 + nf * gmin * gmin) \
            * (inv_denom * inv_denom)
        psum = (_PMAX - _PMIN) * sumq + _PMIN * nf
        scale = jnp.float32(np.float32(np.sqrt(float(_PMAX - _PMIN))))
        out_ref[:, 0:1] = -gmin * inv_denom * scale
        out_ref[:, 1:2] = inv_denom * scale
        out_ref[:, 2:3] = 1.0 / psum
        out_ref[:, 3:4] = psum


def _threefry_xor_pre(x1):
    ks1 = jnp.uint32(1)
    ks2 = jnp.uint32(0x1BD11BDB)

    def r4(x0, x1, rs):
        for r in rs:
            x0 = x0 + x1
            x1 = (x1 << jnp.uint32(r)) | (x1 >> jnp.uint32(32 - r))
            x1 = x1 ^ x0
        return x0, x1

    x0 = x1
    x1 = ((x1 << jnp.uint32(13)) | (x1 >> jnp.uint32(19))) ^ x0
    x0, x1 = r4(x0, x1, (15, 26, 6))
    x0 = x0 + ks1
    x1 = x1 + (ks2 + jnp.uint32(1))
    x0, x1 = r4(x0, x1, (17, 29, 16, 24))
    x0 = x0 + ks2
    x1 = x1 + jnp.uint32(2)
    x0, x1 = r4(x0, x1, (13, 15, 26, 6))
    x1 = x1 + jnp.uint32(4)
    x0, x1 = r4(x0, x1, (17, 29, 16, 24))
    x0 = x0 + ks1
    x1 = x1 + (ks2 + jnp.uint32(4))
    x0, x1 = r4(x0, x1, (13, 15, 26, 6))
    x0 = x0 + ks2
    x1 = x1 + jnp.uint32(5)
    return x0 ^ x1


def _main_kernel(x_ref, stats_ref, probs_ref, idx_ref, accp_ref, accw_ref,
                 acci_ref, *, n_cols, block_w, sub_w):
    c = pl.program_id(0)
    nc = pl.num_programs(0)
    B = x_ref.shape[0]

    c0 = stats_ref[:, 0:1]
    c1 = stats_ref[:, 1:2]
    inv_psum = stats_ref[:, 2:3]

    accp = jnp.where(c == 0, jnp.zeros((B, _LANES), jnp.float32),
                     accp_ref[...])
    accw = jnp.where(c == 0, jnp.ones((B, _LANES), jnp.float32),
                     accw_ref[...])
    acci = jnp.where(c == 0, jnp.zeros((B, _LANES), jnp.uint32),
                     acci_ref[...])

    base = c * block_w
    rowc = jax.lax.broadcasted_iota(jnp.uint32, (B, _LANES), 0) \
        * jnp.uint32(n_cols)
    lane = jax.lax.broadcasted_iota(jnp.uint32, (B, _LANES), 1)
    g0 = rowc + lane + jnp.uint32(base + 1)
    thr = rowc + jnp.uint32(n_cols + 1)

    for s in range(block_w // sub_w):
        off = s * sub_w
        xs = x_ref[:, off:off + sub_w]
        a = jnp.abs(xs)
        t = a * c1 + c0
        p = t * t + _PMIN
        probs_ref[:, off:off + sub_w] = p * inv_psum

        for k in range(sub_w // _LANES):
            fk = g0 + jnp.uint32(off + k * _LANES)
            bits = _threefry_xor_pre(fk)
            fb = (bits >> jnp.uint32(9)) | jnp.uint32(0x3F800000)
            fl = jax.lax.bitcast_convert_type(fb, jnp.float32) \
                - jnp.float32(1.0)
            u = jnp.maximum(_TINY, fl + _TINY)
            w = -jnp.log(u)
            pk = p[:, k * _LANES:(k + 1) * _LANES]
            upd = (pk * accw > accp * w) & (fk < thr)
            accp = jnp.where(upd, pk, accp)
            accw = jnp.where(upd, w, accw)
            acci = jnp.where(upd, fk, acci)

    accp_ref[...] = accp
    accw_ref[...] = accw
    acci_ref[...] = acci

    @pl.when(c == nc - 1)
    def _():
        z = accp / accw
        zmax = jnp.max(z, axis=-1, keepdims=True)
        col = (acci - jnp.uint32(1) - rowc).astype(jnp.int32)
        li = jnp.min(jnp.where(z == zmax, col, jnp.int32(n_cols)),
                     axis=-1, keepdims=True)
        idx_ref[...] = li


def kernel(igrad):
    B, N = igrad.shape
    w1 = 32768
    c1 = pl.cdiv(N, w1)
    stats = pl.pallas_call(
        functools.partial(_stats_kernel, n_cols=N, block_w=w1),
        grid=(c1,),
        in_specs=[pl.BlockSpec((B, w1), lambda c: (0, c))],
        out_specs=pl.BlockSpec((B, 4), lambda c: (0, 0)),
        out_shape=jax.ShapeDtypeStruct((B, 4), jnp.float32),
        scratch_shapes=[pltpu.VMEM((B, _LANES), jnp.float32)
                        for _ in range(4)],
    )(igrad)

    w2 = 65536
    sub_w = 128
    c2 = pl.cdiv(N, w2)
    probs, idx2 = pl.pallas_call(
        functools.partial(_main_kernel, n_cols=N, block_w=w2, sub_w=sub_w),
        grid=(c2,),
        in_specs=[pl.BlockSpec((B, w2), lambda c: (0, c)),
                  pl.BlockSpec((B, 4), lambda c: (0, 0))],
        out_specs=[pl.BlockSpec((B, w2), lambda c: (0, c)),
                   pl.BlockSpec((B, 1), lambda c: (0, 0))],
        out_shape=[jax.ShapeDtypeStruct((B, N), jnp.float32),
                   jax.ShapeDtypeStruct((B, 1), jnp.int32)],
        scratch_shapes=[pltpu.VMEM((B, _LANES), jnp.float32),
                        pltpu.VMEM((B, _LANES), jnp.float32),
                        pltpu.VMEM((B, _LANES), jnp.uint32)],
    )(igrad, stats)
    return probs, idx2.reshape(B)


# speedup vs baseline: 1.3050x; 1.3050x over previous
import functools

import jax
import jax.numpy as jnp
import numpy as np
from jax.experimental import pallas as pl
from jax.experimental.pallas import tpu as pltpu

_PMIN = np.float32(0.1)
_PMAX = np.float32(1.0)
_EPS_D = np.float32(1e-12)
_TINY = np.float32(np.finfo(np.float32).tiny)
_LANES = 128


def _stats_kernel(x_ref, out_ref, mn_ref, mx_ref, s1_ref, s2_ref, *,
                  n_cols, block_w):
    c = pl.program_id(0)
    nc = pl.num_programs(0)
    B = x_ref.shape[0]
    nfold = block_w // _LANES

    accmn = jnp.where(c == 0, jnp.full((B, _LANES), jnp.inf, jnp.float32),
                      mn_ref[...])
    accmx = jnp.where(c == 0, jnp.zeros((B, _LANES), jnp.float32),
                      mx_ref[...])
    accs1 = jnp.where(c == 0, jnp.zeros((B, _LANES), jnp.float32),
                      s1_ref[...])
    accs2 = jnp.where(c == 0, jnp.zeros((B, _LANES), jnp.float32),
                      s2_ref[...])

    base = c * block_w

    def fold(masked):
        amn, amx, a1, a2 = accmn, accmx, accs1, accs2
        lane = jax.lax.broadcasted_iota(jnp.int32, (B, _LANES), 1)
        for k in range(nfold):
            a = jnp.abs(x_ref[:, k * _LANES:(k + 1) * _LANES])
            if masked:
                valid = (base + k * _LANES) + lane < n_cols
                am = jnp.where(valid, a, 0.0)
                amn = jnp.minimum(amn, jnp.where(valid, a, jnp.inf))
            else:
                am = a
                amn = jnp.minimum(amn, a)
            amx = jnp.maximum(amx, am)
            a1 = a1 + am
            a2 = a2 + am * am
        mn_ref[...] = amn
        mx_ref[...] = amx
        s1_ref[...] = a1
        s2_ref[...] = a2

    if block_w * (n_cols // block_w) == n_cols:
        fold(False)
    else:
        @pl.when(c != nc - 1)
        def _():
            fold(False)

        @pl.when(c == nc - 1)
        def _():
            fold(True)

    @pl.when(c == nc - 1)
    def _():
        gmin = jnp.min(mn_ref[...], axis=-1, keepdims=True)
        gmax = jnp.max(mx_ref[...], axis=-1, keepdims=True)
        s1 = jnp.sum(s1_ref[...], axis=-1, keepdims=True)
        s2 = jnp.sum(s2_ref[...], axis=-1, keepdims=True)
        denom = gmax - gmin + _EPS_D
        inv_denom = 1.0 / denom
        nf = jnp.float32(n_cols)
        sumq = (s2 - 2.0 * gmin * s1 + nf * gmin * gmin) \
            * (inv_denom * inv_denom)
        psum = (_PMAX - _PMIN) * sumq + _PMIN * nf
        scale = jnp.float32(np.float32(np.sqrt(float(_PMAX - _PMIN))))
        out_ref[:, 0:1] = -gmin * inv_denom * scale
        out_ref[:, 1:2] = inv_denom * scale
        out_ref[:, 2:3] = 1.0 / psum
        out_ref[:, 3:4] = psum


def _threefry_xor_pre(x1):
    ks1 = jnp.uint32(1)
    ks2 = jnp.uint32(0x1BD11BDB)

    def r4(x0, x1, rs):
        for r in rs:
            x0 = x0 + x1
            x1 = (x1 << jnp.uint32(r)) | (x1 >> jnp.uint32(32 - r))
            x1 = x1 ^ x0
        return x0, x1

    x0 = x1
    x1 = ((x1 << jnp.uint32(13)) | (x1 >> jnp.uint32(19))) ^ x0
    x0, x1 = r4(x0, x1, (15, 26, 6))
    x0 = x0 + ks1
    x1 = x1 + (ks2 + jnp.uint32(1))
    x0, x1 = r4(x0, x1, (17, 29, 16, 24))
    x0 = x0 + ks2
    x1 = x1 + jnp.uint32(2)
    x0, x1 = r4(x0, x1, (13, 15, 26, 6))
    x1 = x1 + jnp.uint32(4)
    x0, x1 = r4(x0, x1, (17, 29, 16, 24))
    x0 = x0 + ks1
    x1 = x1 + (ks2 + jnp.uint32(4))
    x0, x1 = r4(x0, x1, (13, 15, 26, 6))
    x0 = x0 + ks2
    x1 = x1 + jnp.uint32(5)
    return x0 ^ x1


def _main_kernel(x_ref, stats_ref, probs_ref, idx_ref, accp_ref, accw_ref,
                 acci_ref, *, n_cols, block_w, sub_w):
    c = pl.program_id(0)
    nc = pl.num_programs(0)
    B = x_ref.shape[0]

    c0 = stats_ref[:, 0:1]
    c1 = stats_ref[:, 1:2]
    inv_psum = stats_ref[:, 2:3]

    accp = jnp.where(c == 0, jnp.zeros((B, _LANES), jnp.float32),
                     accp_ref[...])
    accw = jnp.where(c == 0, jnp.ones((B, _LANES), jnp.float32),
                     accw_ref[...])
    acci = jnp.where(c == 0, jnp.zeros((B, _LANES), jnp.uint32),
                     acci_ref[...])

    base = c * block_w
    rowc = jax.lax.broadcasted_iota(jnp.uint32, (B, _LANES), 0) \
        * jnp.uint32(n_cols)
    lane = jax.lax.broadcasted_iota(jnp.uint32, (B, _LANES), 1)
    g0 = rowc + lane + jnp.uint32(base + 1)
    thr = rowc + jnp.uint32(n_cols + 1)

    for s in range(block_w // sub_w):
        off = s * sub_w
        xs = x_ref[:, off:off + sub_w]
        a = jnp.abs(xs)
        t = a * c1 + c0
        p = t * t + _PMIN
        probs_ref[:, off:off + sub_w] = p * inv_psum

        for k in range(sub_w // _LANES):
            fk = g0 + jnp.uint32(off + k * _LANES)
            bits = _threefry_xor_pre(fk)
            fb = (bits >> jnp.uint32(9)) | jnp.uint32(0x3F800000)
            fl = jax.lax.bitcast_convert_type(fb, jnp.float32) \
                - jnp.float32(1.0)
            u = jnp.maximum(_TINY, fl + _TINY)
            w = -jnp.log(u)
            pk = p[:, k * _LANES:(k + 1) * _LANES]
            upd = (pk * accw > accp * w) & (fk < thr)
            accp = jnp.where(upd, pk, accp)
            accw = jnp.where(upd, w, accw)
            acci = jnp.where(upd, fk, acci)

    accp_ref[...] = accp
    accw_ref[...] = accw
    acci_ref[...] = acci

    @pl.when(c == nc - 1)
    def _():
        z = accp / accw
        zmax = jnp.max(z, axis=-1, keepdims=True)
        col = (acci - jnp.uint32(1) - rowc).astype(jnp.int32)
        li = jnp.min(jnp.where(z == zmax, col, jnp.int32(n_cols)),
                     axis=-1, keepdims=True)
        idx_ref[...] = li


def kernel(igrad):
    B, N = igrad.shape
    w1 = 32768
    c1 = pl.cdiv(N, w1)
    stats = pl.pallas_call(
        functools.partial(_stats_kernel, n_cols=N, block_w=w1),
        grid=(c1,),
        in_specs=[pl.BlockSpec((B, w1), lambda c: (0, c))],
        out_specs=pl.BlockSpec((B, 4), lambda c: (0, 0)),
        out_shape=jax.ShapeDtypeStruct((B, 4), jnp.float32),
        scratch_shapes=[pltpu.VMEM((B, _LANES), jnp.float32)
                        for _ in range(4)],
    )(igrad)

    w2 = 32768
    sub_w = 128
    c2 = pl.cdiv(N, w2)
    probs, idx2 = pl.pallas_call(
        functools.partial(_main_kernel, n_cols=N, block_w=w2, sub_w=sub_w),
        grid=(c2,),
        in_specs=[pl.BlockSpec((B, w2), lambda c: (0, c)),
                  pl.BlockSpec((B, 4), lambda c: (0, 0))],
        out_specs=[pl.BlockSpec((B, w2), lambda c: (0, c)),
                   pl.BlockSpec((B, 1), lambda c: (0, 0))],
        out_shape=[jax.ShapeDtypeStruct((B, N), jnp.float32),
                   jax.ShapeDtypeStruct((B, 1), jnp.int32)],
        scratch_shapes=[pltpu.VMEM((B, _LANES), jnp.float32),
                        pltpu.VMEM((B, _LANES), jnp.float32),
                        pltpu.VMEM((B, _LANES), jnp.uint32)],
    )(igrad, stats)
    return probs, idx2.reshape(B)
